# Initial kernel scaffold; baseline (speedup 1.0000x reference)
import jax
import jax.numpy as jnp
from jax.experimental import pallas as pl


def kernel(user_table, item_table, W_homo_u, a_homo_u, W_homo_i, a_homo_i, q_hete_u, q_hete_i, W_fuse, user_ids, item_ids, user_neighs, item_neighs):
    raise NotImplementedError("write your pallas kernel here")


# speedup vs baseline: 2.7419x; 2.7419x over previous
import functools

import jax
import jax.numpy as jnp
from jax import lax
from jax.experimental import pallas as pl
from jax.experimental.pallas import tpu as pltpu
from jax.experimental.pallas import tpu_sc as plsc

D = 128
P = 4
N = 16
PN = P * N

_NW = 32
_CHUNK = 128
_T = 64


def _sc_gather(item_table, user_table, idx_item, idx_user):
    n_idx = idx_item.shape[0]
    per_w = n_idx // _NW
    n_chunks = per_w // _CHUNK
    mesh = plsc.VectorSubcoreMesh(core_axis_name="c", subcore_axis_name="s")

    @functools.partial(
        pl.kernel,
        mesh=mesh,
        out_type=[
            jax.ShapeDtypeStruct((n_idx, D), jnp.float32),
            jax.ShapeDtypeStruct((n_idx, D), jnp.float32),
        ],
        scratch_types=[
            pltpu.VMEM((_CHUNK,), jnp.int32),
            pltpu.VMEM((_CHUNK, D), jnp.float32),
            pltpu.SemaphoreType.DMA,
        ],
    )
    def gather_k(item_hbm, user_hbm, idx_i_hbm, idx_u_hbm, out_i, out_u,
                 idx_v, rows_v, sem):
        wid = lax.axis_index("s") * 2 + lax.axis_index("c")
        base = wid * per_w

        def run(table_hbm, idx_hbm, out_hbm):
            def chunk(k, carry):
                off = pl.multiple_of(base + k * _CHUNK, _CHUNK)
                pltpu.sync_copy(idx_hbm.at[pl.ds(off, _CHUNK)], idx_v)
                pltpu.async_copy(table_hbm.at[idx_v], rows_v, sem).wait()
                pltpu.sync_copy(rows_v, out_hbm.at[pl.ds(off, _CHUNK)])
                return carry

            lax.fori_loop(0, n_chunks, chunk, 0)

        run(item_hbm, idx_i_hbm, out_i)
        run(user_hbm, idx_u_hbm, out_u)

    return gather_k(item_table, user_table, idx_item, idx_user)


def _attn_body(nbu_ref, nbv_ref, xu_ref, xv_ref, Wu_ref, au_ref, Wi_ref,
               ai_ref, qu_ref, qi_ref, Wf_ref, out_ref):
    def side(nb_flat, x, W, a, q):
        h = jnp.dot(nb_flat, W, preferred_element_type=jnp.float32)
        xq = jnp.dot(x, W, preferred_element_type=jnp.float32)
        h4 = h.reshape(_T, P, N, D)
        pes = []
        for p in range(P):
            hp = h4[:, p]
            t = jnp.tanh(hp + xq[:, None, :])
            s = jnp.sum(t * a, axis=-1)
            m = jnp.max(s, axis=-1, keepdims=True)
            e = jnp.exp(s - m)
            att = e / jnp.sum(e, axis=-1, keepdims=True)
            pes.append(jnp.sum(att[:, :, None] * hp, axis=1))
        ss = [jnp.sum(jnp.tanh(pe) * q, axis=-1, keepdims=True) for pe in pes]
        m = jnp.maximum(jnp.maximum(ss[0], ss[1]), jnp.maximum(ss[2], ss[3]))
        es = [jnp.exp(sp - m) for sp in ss]
        z = es[0] + es[1] + es[2] + es[3]
        agg = (es[0] * pes[0] + es[1] * pes[1]
               + es[2] * pes[2] + es[3] * pes[3]) / z
        return x + agg

    u = side(nbu_ref[...], xu_ref[...], Wu_ref[...], au_ref[...], qu_ref[...])
    v = side(nbv_ref[...], xv_ref[...], Wi_ref[...], ai_ref[...], qi_ref[...])
    vf = jnp.dot(v, Wf_ref[...], preferred_element_type=jnp.float32)
    out_ref[...] = jnp.sum(u * vf, axis=-1, keepdims=True)


def _tc_attn(g_item, g_user, n_nb, Wu, au, Wi, ai, qu, qi, Wf, B):
    grid = B // _T
    ctr_block0 = n_nb // _T
    nb_spec = pl.BlockSpec((_T * PN, D), lambda i: (i, 0))
    ctr_spec = pl.BlockSpec((_T, D), lambda i: (ctr_block0 + i, 0))
    w_spec = pl.BlockSpec((D, D), lambda i: (0, 0))
    vec_spec = pl.BlockSpec((1, D), lambda i: (0, 0))
    return pl.pallas_call(
        _attn_body,
        grid=(grid,),
        in_specs=[nb_spec, nb_spec, ctr_spec, ctr_spec,
                  w_spec, vec_spec, w_spec, vec_spec,
                  vec_spec, vec_spec, w_spec],
        out_specs=pl.BlockSpec((_T, 1), lambda i: (i, 0)),
        out_shape=jax.ShapeDtypeStruct((B, 1), jnp.float32),
    )(g_item, g_user, g_user, g_item, Wu, au, Wi, ai, qu, qi, Wf)


def kernel(user_table, item_table, W_homo_u, a_homo_u, W_homo_i, a_homo_i,
           q_hete_u, q_hete_i, W_fuse, user_ids, item_ids, user_neighs,
           item_neighs):
    B = user_ids.shape[0]
    n_nb = B * PN
    idx_item = jnp.concatenate(
        [user_neighs.reshape(-1), item_ids]).astype(jnp.int32)
    idx_user = jnp.concatenate(
        [item_neighs.reshape(-1), user_ids]).astype(jnp.int32)
    g_item, g_user = _sc_gather(item_table, user_table, idx_item, idx_user)
    score = _tc_attn(
        g_item, g_user, n_nb,
        W_homo_u, a_homo_u.reshape(1, D), W_homo_i, a_homo_i.reshape(1, D),
        q_hete_u.reshape(1, D), q_hete_i.reshape(1, D), W_fuse, B)
    return score.reshape(B)


# speedup vs baseline: 5.1165x; 1.8661x over previous
import functools

import jax
import jax.numpy as jnp
from jax import lax
from jax.experimental import pallas as pl
from jax.experimental.pallas import tpu as pltpu
from jax.experimental.pallas import tpu_sc as plsc

D = 128
P = 4
N = 16
PN = P * N

_NW = 32
_CHUNK = 128
_T = 128


def _sc_gather(item_table, user_table, idx_item, idx_user):
    n_idx = idx_item.shape[0]
    per_w = n_idx // _NW
    n_chunks = per_w // _CHUNK
    mesh = plsc.VectorSubcoreMesh(core_axis_name="c", subcore_axis_name="s")

    @functools.partial(
        pl.kernel,
        mesh=mesh,
        out_type=[
            jax.ShapeDtypeStruct((n_idx, D), jnp.float32),
            jax.ShapeDtypeStruct((n_idx, D), jnp.float32),
        ],
        scratch_types=[
            pltpu.VMEM((per_w,), jnp.int32),
            pltpu.VMEM((_CHUNK, D), jnp.float32),
            pltpu.VMEM((_CHUNK, D), jnp.float32),
            pltpu.SemaphoreType.DMA,
        ],
    )
    def gather_k(item_hbm, user_hbm, idx_i_hbm, idx_u_hbm, out_i, out_u,
                 idx_all, rows0, rows1, gsem):
        wid = lax.axis_index("s") * 2 + lax.axis_index("c")
        base = wid * per_w
        rows = (rows0, rows1)

        def run(table_hbm, idx_hbm, out_hbm):
            pltpu.sync_copy(idx_hbm.at[pl.ds(pl.multiple_of(base, _CHUNK),
                                             per_w)], idx_all)

            def gath(k, buf):
                off = pl.multiple_of(k * _CHUNK, _CHUNK)
                pltpu.async_copy(table_hbm.at[idx_all.at[pl.ds(off, _CHUNK)]],
                                 buf, gsem)

            def drain(k, buf):
                pltpu.make_async_copy(
                    table_hbm.at[idx_all.at[pl.ds(0, _CHUNK)]],
                    buf, gsem).wait()
                off = pl.multiple_of(base + k * _CHUNK, _CHUNK)
                pltpu.sync_copy(buf, out_hbm.at[pl.ds(off, _CHUNK)])

            gath(0, rows[0])

            def body(j, carry):
                for b in range(2):
                    kk = 2 * j + b
                    gath(kk + 1, rows[1 - b])
                    drain(kk, rows[b])
                return carry

            lax.fori_loop(0, (n_chunks - 1) // 2, body, 0, unroll=False)
            drain(n_chunks - 1, rows[(n_chunks - 1) % 2])

        run(item_hbm, idx_i_hbm, out_i)
        run(user_hbm, idx_u_hbm, out_u)

    return gather_k(item_table, user_table, idx_item, idx_user)


def _attn_body(nbu_ref, nbv_ref, xu_ref, xv_ref, Wu_ref, au_ref, Wi_ref,
               ai_ref, qu_ref, qi_ref, Wf_ref, out_ref):
    def side(nb_flat, x, W, Arep, Qrep):
        h = jnp.dot(nb_flat, W, preferred_element_type=jnp.float32)
        xq = jnp.dot(x, W, preferred_element_type=jnp.float32)
        t = jnp.tanh(h.reshape(_T, PN, D) + xq[:, None, :]).reshape(_T * PN, D)
        e = jnp.exp(jnp.dot(t, Arep, preferred_element_type=jnp.float32))
        num = (e * h).reshape(_T * P, N, D).sum(axis=1)
        den = e.reshape(_T * P, N, D).sum(axis=1)
        pe = num / den
        ep = jnp.exp(jnp.dot(jnp.tanh(pe), Qrep,
                             preferred_element_type=jnp.float32))
        aggn = (ep * pe).reshape(_T, P, D).sum(axis=1)
        aggd = ep.reshape(_T, P, D).sum(axis=1)
        return x + aggn / aggd

    u = side(nbu_ref[...], xu_ref[...], Wu_ref[...], au_ref[...], qu_ref[...])
    v = side(nbv_ref[...], xv_ref[...], Wi_ref[...], ai_ref[...], qi_ref[...])
    vf = jnp.dot(v, Wf_ref[...], preferred_element_type=jnp.float32)
    out_ref[...] = jnp.sum(u * vf, axis=-1, keepdims=True)


def _tc_attn(g_item, g_user, n_nb, Wu, au, Wi, ai, qu, qi, Wf, B):
    grid = B // _T
    ctr_block0 = n_nb // _T
    nb_spec = pl.BlockSpec((_T * PN, D), lambda i: (i, 0))
    ctr_spec = pl.BlockSpec((_T, D), lambda i: (ctr_block0 + i, 0))
    w_spec = pl.BlockSpec((D, D), lambda i: (0, 0))
    return pl.pallas_call(
        _attn_body,
        grid=(grid,),
        in_specs=[nb_spec, nb_spec, ctr_spec, ctr_spec,
                  w_spec, w_spec, w_spec, w_spec,
                  w_spec, w_spec, w_spec],
        out_specs=pl.BlockSpec((_T, 1), lambda i: (i, 0)),
        out_shape=jax.ShapeDtypeStruct((B, 1), jnp.float32),
    )(g_item, g_user, g_user, g_item, Wu, au, Wi, ai, qu, qi, Wf)


def kernel(user_table, item_table, W_homo_u, a_homo_u, W_homo_i, a_homo_i,
           q_hete_u, q_hete_i, W_fuse, user_ids, item_ids, user_neighs,
           item_neighs):
    B = user_ids.shape[0]
    n_nb = B * PN
    idx_item = jnp.concatenate(
        [user_neighs.reshape(-1), item_ids]).astype(jnp.int32)
    idx_user = jnp.concatenate(
        [item_neighs.reshape(-1), user_ids]).astype(jnp.int32)
    g_item, g_user = _sc_gather(item_table, user_table, idx_item, idx_user)
    ones = jnp.ones((1, D), jnp.float32)
    score = _tc_attn(
        g_item, g_user, n_nb,
        W_homo_u, a_homo_u[:, None] * ones, W_homo_i, a_homo_i[:, None] * ones,
        q_hete_u[:, None] * ones, q_hete_i[:, None] * ones, W_fuse, B)
    return score.reshape(B)


# speedup vs baseline: 6.2125x; 1.2142x over previous
import functools

import jax
import jax.numpy as jnp
from jax import lax
from jax.experimental import pallas as pl
from jax.experimental.pallas import tpu as pltpu
from jax.experimental.pallas import tpu_sc as plsc

D = 128
P = 4
N = 16
PN = P * N

_NW = 32
_CHUNK = 128
_T = 128
_S = 4


def _sc_gather(item_table, user_table, nbi_idx, nbu_idx, ci_idx, cu_idx):
    nn = nbi_idx.shape[0]
    nc = ci_idx.shape[0]
    per_w = nn // _NW
    n_chunks = per_w // _CHUNK
    cper_w = nc // _NW
    mesh = plsc.VectorSubcoreMesh(core_axis_name="c", subcore_axis_name="s")

    @functools.partial(
        pl.kernel,
        mesh=mesh,
        out_type=[
            jax.ShapeDtypeStruct((nn, D), jnp.float32),
            jax.ShapeDtypeStruct((nn, D), jnp.float32),
            jax.ShapeDtypeStruct((nc, D), jnp.float32),
            jax.ShapeDtypeStruct((nc, D), jnp.float32),
        ],
        scratch_types=[
            pltpu.VMEM((per_w,), jnp.int32),
            pltpu.VMEM((_CHUNK, D), jnp.float32),
            pltpu.VMEM((_CHUNK, D), jnp.float32),
            pltpu.VMEM((cper_w,), jnp.int32),
            pltpu.VMEM((cper_w, D), jnp.float32),
            pltpu.SemaphoreType.DMA,
            pltpu.SemaphoreType.DMA,
        ],
    )
    def gather_k(item_hbm, user_hbm, nbi_hbm, nbu_hbm, ci_hbm, cu_hbm,
                 out_nbi, out_nbu, out_ci, out_cu,
                 idx_all, rows0, rows1, cidx, crows, gsem, csem):
        wid = lax.axis_index("s") * 2 + lax.axis_index("c")
        base = wid * per_w
        cbase = wid * cper_w
        rows = (rows0, rows1)

        def run(table_hbm, nbidx_hbm, cidx_hbm, out_nb, out_c):
            pltpu.sync_copy(nbidx_hbm.at[pl.ds(pl.multiple_of(base, _CHUNK),
                                               per_w)], idx_all)
            pltpu.sync_copy(cidx_hbm.at[pl.ds(pl.multiple_of(cbase, 8),
                                              cper_w)], cidx)

            def gath(k, buf):
                off = pl.multiple_of(k * _CHUNK, _CHUNK)
                pltpu.async_copy(table_hbm.at[idx_all.at[pl.ds(off, _CHUNK)]],
                                 buf, gsem)

            def drain(k, buf):
                pltpu.make_async_copy(
                    table_hbm.at[idx_all.at[pl.ds(0, _CHUNK)]],
                    buf, gsem).wait()
                off = pl.multiple_of(base + k * _CHUNK, _CHUNK)
                pltpu.sync_copy(buf, out_nb.at[pl.ds(off, _CHUNK)])

            gath(0, rows[0])
            pltpu.async_copy(table_hbm.at[cidx], crows, csem)

            n_pairs = (n_chunks - 1) // 2

            def body(j, carry):
                for b in range(2):
                    kk = 2 * j + b
                    gath(kk + 1, rows[1 - b])
                    drain(kk, rows[b])
                return carry

            lax.fori_loop(0, n_pairs, body, 0, unroll=False)
            if n_chunks % 2 == 0:
                gath(n_chunks - 1, rows[1])
                drain(n_chunks - 2, rows[0])
                drain(n_chunks - 1, rows[1])
            else:
                drain(n_chunks - 1, rows[0])

            pltpu.make_async_copy(table_hbm.at[cidx], crows, csem).wait()
            pltpu.sync_copy(crows,
                            out_c.at[pl.ds(pl.multiple_of(cbase, 8), cper_w)])

        run(item_hbm, nbi_hbm, ci_hbm, out_nbi, out_ci)
        run(user_hbm, nbu_hbm, cu_hbm, out_nbu, out_cu)

    return gather_k(item_table, user_table, nbi_idx, nbu_idx, ci_idx, cu_idx)


def _attn_body(nbu_ref, nbv_ref, xu_ref, xv_ref, Wu_ref, au_ref, Wi_ref,
               ai_ref, qu_ref, qi_ref, Wf_ref, out_ref):
    def side(nb_flat, x, W, Arep, Qrep):
        h = jnp.dot(nb_flat, W, preferred_element_type=jnp.float32)
        xq = jnp.dot(x, W, preferred_element_type=jnp.float32)
        t = jnp.tanh(h.reshape(_T, PN, D) + xq[:, None, :]).reshape(_T * PN, D)
        e = jnp.exp(jnp.dot(t, Arep, preferred_element_type=jnp.float32))
        num = (e * h).reshape(_T * P, N, D).sum(axis=1)
        den = e.reshape(_T * P, N, D).sum(axis=1)
        pe = num / den
        ep = jnp.exp(jnp.dot(jnp.tanh(pe), Qrep,
                             preferred_element_type=jnp.float32))
        aggn = (ep * pe).reshape(_T, P, D).sum(axis=1)
        aggd = ep.reshape(_T, P, D).sum(axis=1)
        return x + aggn / aggd

    u = side(nbu_ref[...], xu_ref[...], Wu_ref[...], au_ref[...], qu_ref[...])
    v = side(nbv_ref[...], xv_ref[...], Wi_ref[...], ai_ref[...], qi_ref[...])
    vf = jnp.dot(v, Wf_ref[...], preferred_element_type=jnp.float32)
    out_ref[...] = jnp.sum(u * vf, axis=-1, keepdims=True)


def _tc_attn(nb_u, nb_v, x_u, x_v, Wu, Au, Wi, Ai, Qu, Qi, Wf):
    Bs = x_u.shape[0]
    grid = Bs // _T
    nb_spec = pl.BlockSpec((_T * PN, D), lambda i: (i, 0))
    ctr_spec = pl.BlockSpec((_T, D), lambda i: (i, 0))
    w_spec = pl.BlockSpec((D, D), lambda i: (0, 0))
    return pl.pallas_call(
        _attn_body,
        grid=(grid,),
        in_specs=[nb_spec, nb_spec, ctr_spec, ctr_spec,
                  w_spec, w_spec, w_spec, w_spec,
                  w_spec, w_spec, w_spec],
        out_specs=pl.BlockSpec((_T, 1), lambda i: (i, 0)),
        out_shape=jax.ShapeDtypeStruct((Bs, 1), jnp.float32),
    )(nb_u, nb_v, x_u, x_v, Wu, Au, Wi, Ai, Qu, Qi, Wf)


def kernel(user_table, item_table, W_homo_u, a_homo_u, W_homo_i, a_homo_i,
           q_hete_u, q_hete_i, W_fuse, user_ids, item_ids, user_neighs,
           item_neighs):
    B = user_ids.shape[0]
    Bs = B // _S
    un = user_neighs.reshape(B, PN).astype(jnp.int32)
    itn = item_neighs.reshape(B, PN).astype(jnp.int32)
    uid = user_ids.astype(jnp.int32)
    iid = item_ids.astype(jnp.int32)
    ones = jnp.ones((1, D), jnp.float32)
    Au = a_homo_u[:, None] * ones
    Ai = a_homo_i[:, None] * ones
    Qu = q_hete_u[:, None] * ones
    Qi = q_hete_i[:, None] * ones

    outs = []
    for s in range(_S):
        sl = slice(s * Bs, (s + 1) * Bs)
        g_nbi, g_nbu, g_ci, g_cu = _sc_gather(
            item_table, user_table,
            un[sl].reshape(-1), itn[sl].reshape(-1), iid[sl], uid[sl])
        outs.append(_tc_attn(g_nbi, g_nbu, g_cu, g_ci,
                             W_homo_u, Au, W_homo_i, Ai, Qu, Qi, W_fuse))
    return jnp.concatenate(outs, axis=0).reshape(B)


# speedup vs baseline: 6.2885x; 1.0122x over previous
import functools

import jax
import jax.numpy as jnp
from jax import lax
from jax.experimental import pallas as pl
from jax.experimental.pallas import tpu as pltpu
from jax.experimental.pallas import tpu_sc as plsc

D = 128
P = 4
N = 16
PN = P * N

_NW = 32
_CHUNK = 128
_T = 128
_S = 4


def _sc_gather(item_table, user_table, nbi_idx, nbu_idx, ci_idx, cu_idx):
    nn = nbi_idx.shape[0]
    nc = ci_idx.shape[0]
    per_w = nn // _NW
    n_chunks = per_w // _CHUNK
    cper_w = nc // _NW
    mesh = plsc.VectorSubcoreMesh(core_axis_name="c", subcore_axis_name="s")

    @functools.partial(
        pl.kernel,
        mesh=mesh,
        out_type=[
            jax.ShapeDtypeStruct((nn, D), jnp.float32),
            jax.ShapeDtypeStruct((nn, D), jnp.float32),
            jax.ShapeDtypeStruct((nc, D), jnp.float32),
            jax.ShapeDtypeStruct((nc, D), jnp.float32),
        ],
        scratch_types=[
            pltpu.VMEM((per_w,), jnp.int32),
            pltpu.VMEM((_CHUNK, D), jnp.float32),
            pltpu.VMEM((_CHUNK, D), jnp.float32),
            pltpu.VMEM((_CHUNK, D), jnp.float32),
            pltpu.VMEM((_CHUNK, D), jnp.float32),
            pltpu.VMEM((cper_w,), jnp.int32),
            pltpu.VMEM((cper_w, D), jnp.float32),
            pltpu.SemaphoreType.DMA,
            pltpu.SemaphoreType.DMA,
            pltpu.SemaphoreType.DMA,
        ],
    )
    def gather_k(item_hbm, user_hbm, nbi_hbm, nbu_hbm, ci_hbm, cu_hbm,
                 out_nbi, out_nbu, out_ci, out_cu,
                 idx_all, rows0, rows1, rows2, rows3, cidx, crows,
                 gsem, wsem, csem):
        wid = lax.axis_index("s") * 2 + lax.axis_index("c")
        base = wid * per_w
        cbase = wid * cper_w
        rows = (rows0, rows1, rows2, rows3)
        depth = 3

        def run(table_hbm, nbidx_hbm, cidx_hbm, out_nb, out_c):
            pltpu.sync_copy(nbidx_hbm.at[pl.ds(pl.multiple_of(base, _CHUNK),
                                               per_w)], idx_all)
            pltpu.sync_copy(cidx_hbm.at[pl.ds(pl.multiple_of(cbase, 8),
                                              cper_w)], cidx)

            def gath(k):
                off = pl.multiple_of(k * _CHUNK, _CHUNK)
                pltpu.async_copy(table_hbm.at[idx_all.at[pl.ds(off, _CHUNK)]],
                                 rows[k % 4], gsem)

            def out_at(k):
                off = pl.multiple_of(base + k * _CHUNK, _CHUNK)
                return out_nb.at[pl.ds(off, _CHUNK)]

            for k in range(depth):
                gath(k)
            pltpu.async_copy(table_hbm.at[cidx], crows, csem)

            n_wsem_waits = 0
            for k in range(n_chunks):
                pltpu.make_async_copy(
                    table_hbm.at[idx_all.at[pl.ds(0, _CHUNK)]],
                    rows[k % 4], gsem).wait()
                pltpu.async_copy(rows[k % 4], out_at(k), wsem)
                nxt = k + depth
                if nxt < n_chunks:
                    if k >= 1:
                        pltpu.make_async_copy(rows[0], out_at(0), wsem).wait()
                        n_wsem_waits += 1
                    gath(nxt)
            for _ in range(n_chunks - n_wsem_waits):
                pltpu.make_async_copy(rows[0], out_at(0), wsem).wait()

            pltpu.make_async_copy(table_hbm.at[cidx], crows, csem).wait()
            pltpu.sync_copy(crows,
                            out_c.at[pl.ds(pl.multiple_of(cbase, 8), cper_w)])

        run(item_hbm, nbi_hbm, ci_hbm, out_nbi, out_ci)
        run(user_hbm, nbu_hbm, cu_hbm, out_nbu, out_cu)

    return gather_k(item_table, user_table, nbi_idx, nbu_idx, ci_idx, cu_idx)


def _attn_body(nbu_ref, nbv_ref, xu_ref, xv_ref, Wu_ref, au_ref, Wi_ref,
               ai_ref, qu_ref, qi_ref, Wf_ref, out_ref):
    def side(nb_flat, x, W, Arep, Qrep):
        h = jnp.dot(nb_flat, W, preferred_element_type=jnp.float32)
        xq = jnp.dot(x, W, preferred_element_type=jnp.float32)
        t = jnp.tanh(h.reshape(_T, PN, D) + xq[:, None, :]).reshape(_T * PN, D)
        e = jnp.exp(jnp.dot(t, Arep, preferred_element_type=jnp.float32))
        num = (e * h).reshape(_T * P, N, D).sum(axis=1)
        den = e.reshape(_T * P, N, D).sum(axis=1)
        pe = num / den
        ep = jnp.exp(jnp.dot(jnp.tanh(pe), Qrep,
                             preferred_element_type=jnp.float32))
        aggn = (ep * pe).reshape(_T, P, D).sum(axis=1)
        aggd = ep.reshape(_T, P, D).sum(axis=1)
        return x + aggn / aggd

    u = side(nbu_ref[...], xu_ref[...], Wu_ref[...], au_ref[...], qu_ref[...])
    v = side(nbv_ref[...], xv_ref[...], Wi_ref[...], ai_ref[...], qi_ref[...])
    vf = jnp.dot(v, Wf_ref[...], preferred_element_type=jnp.float32)
    out_ref[...] = jnp.sum(u * vf, axis=-1, keepdims=True)


def _tc_attn(nb_u, nb_v, x_u, x_v, Wu, Au, Wi, Ai, Qu, Qi, Wf):
    Bs = x_u.shape[0]
    grid = Bs // _T
    nb_spec = pl.BlockSpec((_T * PN, D), lambda i: (i, 0))
    ctr_spec = pl.BlockSpec((_T, D), lambda i: (i, 0))
    w_spec = pl.BlockSpec((D, D), lambda i: (0, 0))
    return pl.pallas_call(
        _attn_body,
        grid=(grid,),
        in_specs=[nb_spec, nb_spec, ctr_spec, ctr_spec,
                  w_spec, w_spec, w_spec, w_spec,
                  w_spec, w_spec, w_spec],
        out_specs=pl.BlockSpec((_T, 1), lambda i: (i, 0)),
        out_shape=jax.ShapeDtypeStruct((Bs, 1), jnp.float32),
    )(nb_u, nb_v, x_u, x_v, Wu, Au, Wi, Ai, Qu, Qi, Wf)


def kernel(user_table, item_table, W_homo_u, a_homo_u, W_homo_i, a_homo_i,
           q_hete_u, q_hete_i, W_fuse, user_ids, item_ids, user_neighs,
           item_neighs):
    B = user_ids.shape[0]
    Bs = B // _S
    un = user_neighs.reshape(B, PN).astype(jnp.int32)
    itn = item_neighs.reshape(B, PN).astype(jnp.int32)
    uid = user_ids.astype(jnp.int32)
    iid = item_ids.astype(jnp.int32)
    ones = jnp.ones((1, D), jnp.float32)
    Au = a_homo_u[:, None] * ones
    Ai = a_homo_i[:, None] * ones
    Qu = q_hete_u[:, None] * ones
    Qi = q_hete_i[:, None] * ones

    outs = []
    for s in range(_S):
        sl = slice(s * Bs, (s + 1) * Bs)
        g_nbi, g_nbu, g_ci, g_cu = _sc_gather(
            item_table, user_table,
            un[sl].reshape(-1), itn[sl].reshape(-1), iid[sl], uid[sl])
        outs.append(_tc_attn(g_nbi, g_nbu, g_cu, g_ci,
                             W_homo_u, Au, W_homo_i, Ai, Qu, Qi, W_fuse))
    return jnp.concatenate(outs, axis=0).reshape(B)


# speedup vs baseline: 6.6123x; 1.0515x over previous
import functools

import jax
import jax.numpy as jnp
from jax import lax
from jax.experimental import pallas as pl
from jax.experimental.pallas import tpu as pltpu
from jax.experimental.pallas import tpu_sc as plsc

D = 128
P = 4
N = 16
PN = P * N

_NW = 32
_CHUNK = 128
_T = 128
_S = 4


def _sc_gather(item_table, user_table, nbi_idx, nbu_idx, ci_idx, cu_idx):
    nn = nbi_idx.shape[0]
    nc = ci_idx.shape[0]
    per_w = nn // _NW
    n_chunks = per_w // _CHUNK
    cper_w = nc // _NW
    mesh = plsc.VectorSubcoreMesh(core_axis_name="c", subcore_axis_name="s")

    @functools.partial(
        pl.kernel,
        mesh=mesh,
        out_type=[
            jax.ShapeDtypeStruct((nn, D), jnp.float32),
            jax.ShapeDtypeStruct((nn, D), jnp.float32),
            jax.ShapeDtypeStruct((nc, D), jnp.float32),
            jax.ShapeDtypeStruct((nc, D), jnp.float32),
        ],
        scratch_types=[
            pltpu.VMEM((per_w,), jnp.int32),
            pltpu.VMEM((_CHUNK, D), jnp.float32),
            pltpu.VMEM((_CHUNK, D), jnp.float32),
            pltpu.VMEM((_CHUNK, D), jnp.float32),
            pltpu.VMEM((_CHUNK, D), jnp.float32),
            pltpu.VMEM((cper_w,), jnp.int32),
            pltpu.VMEM((cper_w, D), jnp.float32),
            pltpu.SemaphoreType.DMA,
            pltpu.SemaphoreType.DMA,
            pltpu.SemaphoreType.DMA,
        ],
    )
    def gather_k(item_hbm, user_hbm, nbi_hbm, nbu_hbm, ci_hbm, cu_hbm,
                 out_nbi, out_nbu, out_ci, out_cu,
                 idx_all, rows0, rows1, rows2, rows3, cidx, crows,
                 gsem, wsem, csem):
        wid = lax.axis_index("s") * 2 + lax.axis_index("c")
        base = wid * per_w
        cbase = wid * cper_w
        rows = (rows0, rows1, rows2, rows3)
        depth = 3

        def run(table_hbm, nbidx_hbm, cidx_hbm, out_nb, out_c):
            pltpu.sync_copy(nbidx_hbm.at[pl.ds(pl.multiple_of(base, _CHUNK),
                                               per_w)], idx_all)
            pltpu.sync_copy(cidx_hbm.at[pl.ds(pl.multiple_of(cbase, 8),
                                              cper_w)], cidx)

            def gath(k):
                off = pl.multiple_of(k * _CHUNK, _CHUNK)
                pltpu.async_copy(table_hbm.at[idx_all.at[pl.ds(off, _CHUNK)]],
                                 rows[k % 4], gsem)

            def out_at(k):
                off = pl.multiple_of(base + k * _CHUNK, _CHUNK)
                return out_nb.at[pl.ds(off, _CHUNK)]

            for k in range(depth):
                gath(k)
            pltpu.async_copy(table_hbm.at[cidx], crows, csem)

            n_wsem_waits = 0
            for k in range(n_chunks):
                pltpu.make_async_copy(
                    table_hbm.at[idx_all.at[pl.ds(0, _CHUNK)]],
                    rows[k % 4], gsem).wait()
                pltpu.async_copy(rows[k % 4], out_at(k), wsem)
                nxt = k + depth
                if nxt < n_chunks:
                    if k >= 1:
                        pltpu.make_async_copy(rows[0], out_at(0), wsem).wait()
                        n_wsem_waits += 1
                    gath(nxt)
            for _ in range(n_chunks - n_wsem_waits):
                pltpu.make_async_copy(rows[0], out_at(0), wsem).wait()

            pltpu.make_async_copy(table_hbm.at[cidx], crows, csem).wait()
            pltpu.sync_copy(crows,
                            out_c.at[pl.ds(pl.multiple_of(cbase, 8), cper_w)])

        run(item_hbm, nbi_hbm, ci_hbm, out_nbi, out_ci)
        run(user_hbm, nbu_hbm, cu_hbm, out_nbu, out_cu)

    return gather_k(item_table, user_table, nbi_idx, nbu_idx, ci_idx, cu_idx)


def _attn_body(nbu_ref, nbv_ref, xu_ref, xv_ref, Wu_ref, au_ref, Wi_ref,
               ai_ref, qu_ref, qi_ref, Wf_ref, out_ref):
    def side(nb3, x, W, Arep, Qrep):
        h = jnp.dot(nb3.reshape(PN * _T, D), W,
                    preferred_element_type=jnp.float32)
        xq = jnp.dot(x, W, preferred_element_type=jnp.float32)
        t = jnp.tanh(h.reshape(PN, _T, D) + xq[None]).reshape(PN * _T, D)
        e = jnp.exp(jnp.dot(t, Arep, preferred_element_type=jnp.float32))
        num = (e * h).reshape(P, N, _T, D).sum(axis=1)
        den = e.reshape(P, N, _T, D).sum(axis=1)
        pe = num / den
        ep = jnp.exp(jnp.dot(jnp.tanh(pe).reshape(P * _T, D), Qrep,
                             preferred_element_type=jnp.float32))
        ep = ep.reshape(P, _T, D)
        aggn = (ep * pe).sum(axis=0)
        aggd = ep.sum(axis=0)
        return x + aggn / aggd

    u = side(nbu_ref[...], xu_ref[...], Wu_ref[...], au_ref[...], qu_ref[...])
    v = side(nbv_ref[...], xv_ref[...], Wi_ref[...], ai_ref[...], qi_ref[...])
    vf = jnp.dot(v, Wf_ref[...], preferred_element_type=jnp.float32)
    out_ref[...] = jnp.sum(u * vf, axis=-1, keepdims=True)


def _tc_attn(nb_u, nb_v, x_u, x_v, Wu, Au, Wi, Ai, Qu, Qi, Wf):
    Bs = x_u.shape[0]
    grid = Bs // _T
    nb_u = nb_u.reshape(PN, Bs, D)
    nb_v = nb_v.reshape(PN, Bs, D)
    nb_spec = pl.BlockSpec((PN, _T, D), lambda i: (0, i, 0))
    ctr_spec = pl.BlockSpec((_T, D), lambda i: (i, 0))
    w_spec = pl.BlockSpec((D, D), lambda i: (0, 0))
    return pl.pallas_call(
        _attn_body,
        grid=(grid,),
        in_specs=[nb_spec, nb_spec, ctr_spec, ctr_spec,
                  w_spec, w_spec, w_spec, w_spec,
                  w_spec, w_spec, w_spec],
        out_specs=pl.BlockSpec((_T, 1), lambda i: (i, 0)),
        out_shape=jax.ShapeDtypeStruct((Bs, 1), jnp.float32),
    )(nb_u, nb_v, x_u, x_v, Wu, Au, Wi, Ai, Qu, Qi, Wf)


def kernel(user_table, item_table, W_homo_u, a_homo_u, W_homo_i, a_homo_i,
           q_hete_u, q_hete_i, W_fuse, user_ids, item_ids, user_neighs,
           item_neighs):
    B = user_ids.shape[0]
    Bs = B // _S
    un = user_neighs.astype(jnp.int32).transpose(1, 2, 0)
    itn = item_neighs.astype(jnp.int32).transpose(1, 2, 0)
    uid = user_ids.astype(jnp.int32)
    iid = item_ids.astype(jnp.int32)
    ones = jnp.ones((1, D), jnp.float32)
    Au = a_homo_u[:, None] * ones
    Ai = a_homo_i[:, None] * ones
    Qu = q_hete_u[:, None] * ones
    Qi = q_hete_i[:, None] * ones

    outs = []
    for s in range(_S):
        sl = slice(s * Bs, (s + 1) * Bs)
        g_nbi, g_nbu, g_ci, g_cu = _sc_gather(
            item_table, user_table,
            un[:, :, sl].reshape(-1), itn[:, :, sl].reshape(-1),
            iid[sl], uid[sl])
        outs.append(_tc_attn(g_nbi, g_nbu, g_cu, g_ci,
                             W_homo_u, Au, W_homo_i, Ai, Qu, Qi, W_fuse))
    return jnp.concatenate(outs, axis=0).reshape(B)
